# Initial kernel scaffold; baseline (speedup 1.0000x reference)
import jax
import jax.numpy as jnp
from jax.experimental import pallas as pl


def kernel(x, emb_table, W, b):
    raise NotImplementedError("write your pallas kernel here")


# speedup vs baseline: 8.4204x; 8.4204x over previous
import functools

import jax
import jax.numpy as jnp
from jax import lax
from jax.experimental import pallas as pl
from jax.experimental.pallas import tpu as pltpu
from jax.experimental.pallas import tpu_sc as plsc

VOCAB = 1000000
EMBED_DIM = 16
BATCH = 16384
HIST = 200

NC = 2
NS = 16
NW = NC * NS
ROWS_PER_W = BATCH // NW
CHUNK = 16
N_CHUNKS = ROWS_PER_W // CHUNK
IDX_PER_CHUNK = CHUNK * HIST
GATHER_BLK = 128
N_GATHERS = IDX_PER_CHUNK // GATHER_BLK

_ROUND_MAGIC = 12582912.0


def _shuffle(v, idx):
    dnums = lax.GatherDimensionNumbers(
        offset_dims=(), collapsed_slice_dims=(0,), start_index_map=(0,))
    return lax.gather(v, idx[:, None], dnums, (1,),
                      mode=lax.GatherScatterMode.PROMISE_IN_BOUNDS)


def _hsum(v, rot_idx):
    for k, idx in rot_idx:
        v = v + _shuffle(v, idx)
    return v


def _make_kernel():
    mesh = plsc.VectorSubcoreMesh(
        core_axis_name="c", subcore_axis_name="s", num_cores=NC, num_subcores=NS
    )

    @functools.partial(
        pl.kernel,
        out_type=jax.ShapeDtypeStruct((BATCH,), jnp.float32),
        mesh=mesh,
        scratch_types=[
            pltpu.VMEM((IDX_PER_CHUNK,), jnp.int32),
            pltpu.VMEM((IDX_PER_CHUNK, EMBED_DIM), jnp.float32),
            pltpu.VMEM((16,), jnp.float32),
            pltpu.VMEM((16,), jnp.float32),
            pltpu.VMEM((ROWS_PER_W,), jnp.float32),
            pltpu.SemaphoreType.DMA,
        ],
        compiler_params=pltpu.CompilerParams(use_tc_tiling_on_sc=False),
    )
    def kern(x_hbm, tab_hbm, w_hbm, b_hbm, out_hbm,
             idx_v, rows_v, w_v, b_v, out_v, sem):
        wid = lax.axis_index("s") * NC + lax.axis_index("c")

        pltpu.sync_copy(w_hbm, w_v)
        pltpu.sync_copy(b_hbm, b_v)

        lane = lax.iota(jnp.int32, 16)
        w_vec = w_v[...]
        bias = b_v[...]
        rot_idx = [(k, (lane + k) % 16) for k in (8, 4, 2, 1)]

        def chunk_body(c, carry):
            base = wid * ROWS_PER_W + c * CHUNK
            pltpu.sync_copy(x_hbm.at[pl.ds(base * HIST, IDX_PER_CHUNK)], idx_v)
            copies = []
            for k in range(N_GATHERS):
                copies.append(pltpu.async_copy(
                    tab_hbm.at[idx_v.at[pl.ds(k * GATHER_BLK, GATHER_BLK)]],
                    rows_v.at[pl.ds(k * GATHER_BLK, GATHER_BLK), :],
                    sem))
            for cp in copies:
                cp.wait()

            y = jnp.zeros((16,), jnp.float32)
            for i in range(CHUNK):
                def red_body(jj, accs):
                    a0, a1, a2, a3 = accs
                    r0 = i * HIST + jj * 8
                    a0 = a0 + rows_v[r0 + 0, :] + rows_v[r0 + 4, :]
                    a1 = a1 + rows_v[r0 + 1, :] + rows_v[r0 + 5, :]
                    a2 = a2 + rows_v[r0 + 2, :] + rows_v[r0 + 6, :]
                    a3 = a3 + rows_v[r0 + 3, :] + rows_v[r0 + 7, :]
                    return (a0, a1, a2, a3)
                z = jnp.zeros((16,), jnp.float32)
                a0, a1, a2, a3 = lax.fori_loop(
                    0, HIST // 8, red_body, (z, z, z, z))
                acc = (a0 + a1) + (a2 + a3)
                h = _hsum(acc * w_vec, rot_idx)
                y = jnp.where(lane == i, h, y)

            y = y / jnp.float32(HIST) + bias
            s = 1.0 / (1.0 + jnp.exp(-y))
            t = s * 10000.0
            r = (t + _ROUND_MAGIC) - _ROUND_MAGIC
            out_v[pl.ds(c * CHUNK, CHUNK)] = r / 10000.0
            return carry

        lax.fori_loop(0, N_CHUNKS, chunk_body, 0)
        pltpu.sync_copy(out_v, out_hbm.at[pl.ds(wid * ROWS_PER_W, ROWS_PER_W)])

    return kern


_kern = _make_kernel()


def kernel(x, emb_table, W, b):
    x_flat = x.reshape(BATCH * HIST).astype(jnp.int32)
    w_vec = W.reshape(EMBED_DIM).astype(jnp.float32)
    b_vec = jnp.broadcast_to(b.astype(jnp.float32), (16,))
    out = _kern(x_flat, emb_table, w_vec, b_vec)
    return out.reshape(BATCH, 1)


# speedup vs baseline: 19.4700x; 2.3122x over previous
import functools

import jax
import jax.numpy as jnp
from jax import lax
from jax.experimental import pallas as pl
from jax.experimental.pallas import tpu as pltpu
from jax.experimental.pallas import tpu_sc as plsc

VOCAB = 1000000
EMBED_DIM = 16
BATCH = 16384
HIST = 200

NC = 2
NS = 16
NW = NC * NS
ROWS_PER_W = BATCH // NW
CHUNK = 16
N_CHUNKS = ROWS_PER_W // CHUNK
IDX_PER_CHUNK = CHUNK * HIST
GATHER_BLK = 128
N_GATHERS = IDX_PER_CHUNK // GATHER_BLK

_ROUND_MAGIC = 12582912.0

TC_BLK = 65536
TC_GRID = (VOCAB + TC_BLK - 1) // TC_BLK


def _tc_p_body(t_ref, w_ref, p_ref):
    p_ref[...] = jnp.sum(t_ref[...] * w_ref[...], axis=0)


_tc_p = pl.pallas_call(
    _tc_p_body,
    grid=(TC_GRID,),
    in_specs=[
        pl.BlockSpec((EMBED_DIM, TC_BLK), lambda i: (0, i)),
        pl.BlockSpec((EMBED_DIM, 1), lambda i: (0, 0)),
    ],
    out_specs=pl.BlockSpec((TC_BLK,), lambda i: (i,)),
    out_shape=jax.ShapeDtypeStruct((VOCAB,), jnp.float32),
)


def _make_sc_kernel():
    mesh = plsc.VectorSubcoreMesh(
        core_axis_name="c", subcore_axis_name="s", num_cores=NC, num_subcores=NS
    )

    @functools.partial(
        pl.kernel,
        out_type=jax.ShapeDtypeStruct((BATCH,), jnp.float32),
        mesh=mesh,
        scratch_types=[
            pltpu.VMEM((IDX_PER_CHUNK,), jnp.int32),
            pltpu.VMEM((IDX_PER_CHUNK,), jnp.float32),
            pltpu.VMEM((16,), jnp.float32),
            pltpu.VMEM((ROWS_PER_W,), jnp.float32),
            pltpu.SemaphoreType.DMA,
        ],
        compiler_params=pltpu.CompilerParams(use_tc_tiling_on_sc=False),
    )
    def kern(xr_hbm, p_hbm, b_hbm, out_hbm, idx_v, pg_v, b_v, out_v, sem):
        wid = lax.axis_index("s") * NC + lax.axis_index("c")
        pltpu.sync_copy(b_hbm, b_v)
        bias = b_v[...]

        def chunk_body(c, carry):
            g = wid * N_CHUNKS + c
            pltpu.sync_copy(xr_hbm.at[pl.ds(g * IDX_PER_CHUNK, IDX_PER_CHUNK)],
                            idx_v)
            copies = []
            for k in range(N_GATHERS):
                copies.append(pltpu.async_copy(
                    p_hbm.at[idx_v.at[pl.ds(k * GATHER_BLK, GATHER_BLK)]],
                    pg_v.at[pl.ds(k * GATHER_BLK, GATHER_BLK)],
                    sem))
            for cp in copies:
                cp.wait()

            def red_body(jj, accs):
                a0, a1, a2, a3 = accs
                r0 = jj * 128
                a0 = a0 + pg_v[pl.ds(r0, 16)] + pg_v[pl.ds(r0 + 64, 16)]
                a1 = a1 + pg_v[pl.ds(r0 + 16, 16)] + pg_v[pl.ds(r0 + 80, 16)]
                a2 = a2 + pg_v[pl.ds(r0 + 32, 16)] + pg_v[pl.ds(r0 + 96, 16)]
                a3 = a3 + pg_v[pl.ds(r0 + 48, 16)] + pg_v[pl.ds(r0 + 112, 16)]
                return (a0, a1, a2, a3)
            z = jnp.zeros((16,), jnp.float32)
            a0, a1, a2, a3 = lax.fori_loop(0, HIST // 8, red_body, (z, z, z, z))
            acc = (a0 + a1) + (a2 + a3)

            y = acc / jnp.float32(HIST) + bias
            s = 1.0 / (1.0 + jnp.exp(-y))
            t = s * 10000.0
            r = (t + _ROUND_MAGIC) - _ROUND_MAGIC
            out_v[pl.ds(c * CHUNK, CHUNK)] = r / 10000.0
            return carry

        lax.fori_loop(0, N_CHUNKS, chunk_body, 0)
        pltpu.sync_copy(out_v, out_hbm.at[pl.ds(wid * ROWS_PER_W, ROWS_PER_W)])

    return kern


_sc_kern = _make_sc_kernel()


def kernel(x, emb_table, W, b):
    p = _tc_p(emb_table.T, W.reshape(EMBED_DIM, 1))
    x_r = (x.reshape(BATCH // CHUNK, CHUNK, HIST)
             .transpose(0, 2, 1)
             .reshape(BATCH * HIST)
             .astype(jnp.int32))
    b_vec = jnp.broadcast_to(b.astype(jnp.float32), (16,))
    out = _sc_kern(x_r, p, b_vec)
    return out.reshape(BATCH, 1)


# speedup vs baseline: 28.9920x; 1.4891x over previous
import functools

import jax
import jax.numpy as jnp
from jax import lax
from jax.experimental import pallas as pl
from jax.experimental.pallas import tpu as pltpu
from jax.experimental.pallas import tpu_sc as plsc

VOCAB = 1000000
EMBED_DIM = 16
BATCH = 16384
HIST = 200

NC = 2
NS = 16
NW = NC * NS
ROWS_PER_W = BATCH // NW
CHUNK = 16
N_CHUNKS = ROWS_PER_W // CHUNK
IDX_PER_CHUNK = CHUNK * HIST
GATHER_BLK = 128
N_GATHERS = IDX_PER_CHUNK // GATHER_BLK

_ROUND_MAGIC = 12582912.0

TC_BLK = 65536
TC_GRID = (VOCAB + TC_BLK - 1) // TC_BLK


def _tc_p_body(t_ref, w_ref, p_ref):
    p_ref[...] = jnp.sum(t_ref[...] * w_ref[...], axis=0)


_tc_p = pl.pallas_call(
    _tc_p_body,
    grid=(TC_GRID,),
    in_specs=[
        pl.BlockSpec((EMBED_DIM, TC_BLK), lambda i: (0, i)),
        pl.BlockSpec((EMBED_DIM, 1), lambda i: (0, 0)),
    ],
    out_specs=pl.BlockSpec((TC_BLK,), lambda i: (i,)),
    out_shape=jax.ShapeDtypeStruct((VOCAB,), jnp.float32),
)


def _make_sc_kernel():
    mesh = plsc.VectorSubcoreMesh(
        core_axis_name="c", subcore_axis_name="s", num_cores=NC, num_subcores=NS
    )

    @functools.partial(
        pl.kernel,
        out_type=jax.ShapeDtypeStruct((BATCH,), jnp.float32),
        mesh=mesh,
        scratch_types=[
            pltpu.VMEM((IDX_PER_CHUNK,), jnp.int32),
            pltpu.VMEM((IDX_PER_CHUNK,), jnp.float32),
            pltpu.VMEM((16,), jnp.float32),
            pltpu.VMEM((ROWS_PER_W,), jnp.float32),
            pltpu.VMEM_SHARED((VOCAB,), jnp.float32),
            pltpu.SemaphoreType.DMA,
        ],
        compiler_params=pltpu.CompilerParams(use_tc_tiling_on_sc=False),
    )
    def kern(xr_hbm, p_hbm, b_hbm, out_hbm, idx_v, pg_v, b_v, out_v, p_sh, sem):
        wid = lax.axis_index("s") * NC + lax.axis_index("c")
        sid = lax.axis_index("s")
        pltpu.sync_copy(b_hbm, b_v)
        bias = b_v[...]

        STAGE = VOCAB // 8

        @pl.when(sid < 8)
        def _stage():
            pltpu.sync_copy(p_hbm.at[pl.ds(sid * STAGE, STAGE)],
                            p_sh.at[pl.ds(sid * STAGE, STAGE)])

        plsc.subcore_barrier()

        def chunk_body(c, carry):
            g = wid * N_CHUNKS + c
            pltpu.sync_copy(xr_hbm.at[pl.ds(g * IDX_PER_CHUNK, IDX_PER_CHUNK)],
                            idx_v)
            copies = []
            for k in range(N_GATHERS):
                copies.append(pltpu.async_copy(
                    p_sh.at[idx_v.at[pl.ds(k * GATHER_BLK, GATHER_BLK)]],
                    pg_v.at[pl.ds(k * GATHER_BLK, GATHER_BLK)],
                    sem))
            for cp in copies:
                cp.wait()

            def red_body(jj, accs):
                a0, a1, a2, a3 = accs
                r0 = jj * 128
                a0 = a0 + pg_v[pl.ds(r0, 16)] + pg_v[pl.ds(r0 + 64, 16)]
                a1 = a1 + pg_v[pl.ds(r0 + 16, 16)] + pg_v[pl.ds(r0 + 80, 16)]
                a2 = a2 + pg_v[pl.ds(r0 + 32, 16)] + pg_v[pl.ds(r0 + 96, 16)]
                a3 = a3 + pg_v[pl.ds(r0 + 48, 16)] + pg_v[pl.ds(r0 + 112, 16)]
                return (a0, a1, a2, a3)
            z = jnp.zeros((16,), jnp.float32)
            a0, a1, a2, a3 = lax.fori_loop(0, HIST // 8, red_body, (z, z, z, z))
            acc = (a0 + a1) + (a2 + a3)

            y = acc / jnp.float32(HIST) + bias
            s = 1.0 / (1.0 + jnp.exp(-y))
            t = s * 10000.0
            r = (t + _ROUND_MAGIC) - _ROUND_MAGIC
            out_v[pl.ds(c * CHUNK, CHUNK)] = r / 10000.0
            return carry

        lax.fori_loop(0, N_CHUNKS, chunk_body, 0)
        pltpu.sync_copy(out_v, out_hbm.at[pl.ds(wid * ROWS_PER_W, ROWS_PER_W)])

    return kern


_sc_kern = _make_sc_kernel()


def kernel(x, emb_table, W, b):
    p = _tc_p(emb_table.T, W.reshape(EMBED_DIM, 1))
    x_r = (x.reshape(BATCH // CHUNK, CHUNK, HIST)
             .transpose(0, 2, 1)
             .reshape(BATCH * HIST)
             .astype(jnp.int32))
    b_vec = jnp.broadcast_to(b.astype(jnp.float32), (16,))
    out = _sc_kern(x_r, p, b_vec)
    return out.reshape(BATCH, 1)


# speedup vs baseline: 33.1082x; 1.1420x over previous
import functools

import jax
import jax.numpy as jnp
from jax import lax
from jax.experimental import pallas as pl
from jax.experimental.pallas import tpu as pltpu
from jax.experimental.pallas import tpu_sc as plsc

VOCAB = 1000000
EMBED_DIM = 16
BATCH = 16384
HIST = 200

NC = 2
NS = 16
NW = NC * NS
ROWS_PER_W = BATCH // NW
CHUNK = 16
N_CHUNKS = ROWS_PER_W // CHUNK
N_PAIRS = N_CHUNKS // 2
IDX_PER_CHUNK = CHUNK * HIST
GATHER_BLK = 128
N_GATHERS = IDX_PER_CHUNK // GATHER_BLK

_ROUND_MAGIC = 12582912.0

TC_BLK = 65536
TC_GRID = (VOCAB + TC_BLK - 1) // TC_BLK


def _tc_p_body(t_ref, w_ref, p_ref):
    p_ref[...] = jnp.sum(t_ref[...] * w_ref[...], axis=0)


_tc_p = pl.pallas_call(
    _tc_p_body,
    grid=(TC_GRID,),
    in_specs=[
        pl.BlockSpec((EMBED_DIM, TC_BLK), lambda i: (0, i)),
        pl.BlockSpec((EMBED_DIM, 1), lambda i: (0, 0)),
    ],
    out_specs=pl.BlockSpec((TC_BLK,), lambda i: (i,)),
    out_shape=jax.ShapeDtypeStruct((VOCAB,), jnp.float32),
)


def _make_sc_kernel():
    mesh = plsc.VectorSubcoreMesh(
        core_axis_name="c", subcore_axis_name="s", num_cores=NC, num_subcores=NS
    )

    @functools.partial(
        pl.kernel,
        out_type=jax.ShapeDtypeStruct((BATCH,), jnp.float32),
        mesh=mesh,
        scratch_types=[
            pltpu.VMEM((IDX_PER_CHUNK,), jnp.int32),
            pltpu.VMEM((IDX_PER_CHUNK,), jnp.int32),
            pltpu.VMEM((IDX_PER_CHUNK,), jnp.float32),
            pltpu.VMEM((IDX_PER_CHUNK,), jnp.float32),
            pltpu.VMEM((16,), jnp.float32),
            pltpu.VMEM((ROWS_PER_W,), jnp.float32),
            pltpu.VMEM_SHARED((VOCAB,), jnp.float32),
            pltpu.SemaphoreType.DMA,
            pltpu.SemaphoreType.DMA,
            pltpu.SemaphoreType.DMA,
            pltpu.SemaphoreType.DMA,
        ],
        compiler_params=pltpu.CompilerParams(use_tc_tiling_on_sc=False),
    )
    def kern(xr_hbm, p_hbm, b_hbm, out_hbm,
             idx_v0, idx_v1, pg_v0, pg_v1, b_v, out_v, p_sh,
             sem_i0, sem_i1, sem_g0, sem_g1):
        wid = lax.axis_index("s") * NC + lax.axis_index("c")
        sid = lax.axis_index("s")
        gbase = wid * N_CHUNKS
        pltpu.sync_copy(b_hbm, b_v)
        bias = b_v[...]

        STAGE = VOCAB // 8

        @pl.when(sid < 8)
        def _stage():
            pltpu.sync_copy(p_hbm.at[pl.ds(sid * STAGE, STAGE)],
                            p_sh.at[pl.ds(sid * STAGE, STAGE)])

        plsc.subcore_barrier()

        def start_idx(g, ibuf, sem):
            pltpu.async_copy(
                xr_hbm.at[pl.ds(g * IDX_PER_CHUNK, IDX_PER_CHUNK)], ibuf, sem)

        def wait_idx(ibuf, sem):
            pltpu.make_async_copy(
                xr_hbm.at[pl.ds(0, IDX_PER_CHUNK)], ibuf, sem).wait()

        def fire_gathers(ibuf, gbuf, sem):
            for k in range(N_GATHERS):
                pltpu.async_copy(
                    p_sh.at[ibuf.at[pl.ds(k * GATHER_BLK, GATHER_BLK)]],
                    gbuf.at[pl.ds(k * GATHER_BLK, GATHER_BLK)],
                    sem)

        def wait_gathers(gbuf, sem):
            pltpu.make_async_copy(
                p_hbm.at[pl.ds(0, IDX_PER_CHUNK)], gbuf, sem).wait()

        def reduce_store(gbuf, c):
            def red_body(jj, accs):
                a0, a1, a2, a3 = accs
                r0 = jj * 128
                a0 = a0 + gbuf[pl.ds(r0, 16)] + gbuf[pl.ds(r0 + 64, 16)]
                a1 = a1 + gbuf[pl.ds(r0 + 16, 16)] + gbuf[pl.ds(r0 + 80, 16)]
                a2 = a2 + gbuf[pl.ds(r0 + 32, 16)] + gbuf[pl.ds(r0 + 96, 16)]
                a3 = a3 + gbuf[pl.ds(r0 + 48, 16)] + gbuf[pl.ds(r0 + 112, 16)]
                return (a0, a1, a2, a3)
            z = jnp.zeros((16,), jnp.float32)
            a0, a1, a2, a3 = lax.fori_loop(0, HIST // 8, red_body, (z, z, z, z))
            acc = (a0 + a1) + (a2 + a3)
            y = acc / jnp.float32(HIST) + bias
            s = 1.0 / (1.0 + jnp.exp(-y))
            t = s * 10000.0
            r = (t + _ROUND_MAGIC) - _ROUND_MAGIC
            out_v[pl.ds(c * CHUNK, CHUNK)] = r / 10000.0

        start_idx(gbase, idx_v0, sem_i0)
        wait_idx(idx_v0, sem_i0)
        fire_gathers(idx_v0, pg_v0, sem_g0)
        start_idx(gbase + 1, idx_v1, sem_i1)

        def pair_body(t, carry):
            a = 2 * t
            b = 2 * t + 1
            wait_idx(idx_v1, sem_i1)
            wait_gathers(pg_v0, sem_g0)
            fire_gathers(idx_v1, pg_v1, sem_g1)

            @pl.when(t < N_PAIRS - 1)
            def _():
                start_idx(gbase + a + 2, idx_v0, sem_i0)

            reduce_store(pg_v0, a)
            wait_gathers(pg_v1, sem_g1)

            @pl.when(t < N_PAIRS - 1)
            def _():
                wait_idx(idx_v0, sem_i0)
                fire_gathers(idx_v0, pg_v0, sem_g0)
                start_idx(gbase + b + 2, idx_v1, sem_i1)

            reduce_store(pg_v1, b)
            return carry

        lax.fori_loop(0, N_PAIRS, pair_body, 0)
        pltpu.sync_copy(out_v, out_hbm.at[pl.ds(wid * ROWS_PER_W, ROWS_PER_W)])

    return kern


_sc_kern = _make_sc_kernel()


def kernel(x, emb_table, W, b):
    p = _tc_p(emb_table.T, W.reshape(EMBED_DIM, 1))
    x_r = (x.reshape(BATCH // CHUNK, CHUNK, HIST)
             .transpose(0, 2, 1)
             .reshape(BATCH * HIST)
             .astype(jnp.int32))
    b_vec = jnp.broadcast_to(b.astype(jnp.float32), (16,))
    out = _sc_kern(x_r, p, b_vec)
    return out.reshape(BATCH, 1)


# speedup vs baseline: 33.2913x; 1.0055x over previous
import functools

import jax
import jax.numpy as jnp
from jax import lax
from jax.experimental import pallas as pl
from jax.experimental.pallas import tpu as pltpu
from jax.experimental.pallas import tpu_sc as plsc

VOCAB = 1000000
EMBED_DIM = 16
BATCH = 16384
HIST = 200

NC = 2
NS = 16
NW = NC * NS
ROWS_PER_W = BATCH // NW
CHUNK = 16
N_CHUNKS = ROWS_PER_W // CHUNK
N_PAIRS = N_CHUNKS // 2
IDX_PER_CHUNK = CHUNK * HIST
GATHER_BLK = 128
N_GATHERS = IDX_PER_CHUNK // GATHER_BLK

_ROUND_MAGIC = 12582912.0

TC_BLK = 65536
TC_GRID = (VOCAB + TC_BLK - 1) // TC_BLK


def _tc_p_body(t_ref, w_ref, p_ref):
    p_ref[...] = jnp.sum(t_ref[...] * w_ref[...], axis=0)


_tc_p = pl.pallas_call(
    _tc_p_body,
    grid=(TC_GRID,),
    in_specs=[
        pl.BlockSpec((EMBED_DIM, TC_BLK), lambda i: (0, i)),
        pl.BlockSpec((EMBED_DIM, 1), lambda i: (0, 0)),
    ],
    out_specs=pl.BlockSpec((TC_BLK,), lambda i: (i,)),
    out_shape=jax.ShapeDtypeStruct((VOCAB,), jnp.float32),
)


def _make_sc_kernel():
    mesh = plsc.VectorSubcoreMesh(
        core_axis_name="c", subcore_axis_name="s", num_cores=NC, num_subcores=NS
    )

    @functools.partial(
        pl.kernel,
        out_type=jax.ShapeDtypeStruct((BATCH,), jnp.float32),
        mesh=mesh,
        scratch_types=[
            pltpu.VMEM((IDX_PER_CHUNK,), jnp.int32),
            pltpu.VMEM((IDX_PER_CHUNK,), jnp.int32),
            pltpu.VMEM((IDX_PER_CHUNK,), jnp.float32),
            pltpu.VMEM((IDX_PER_CHUNK,), jnp.float32),
            pltpu.VMEM((16,), jnp.float32),
            pltpu.VMEM((ROWS_PER_W,), jnp.float32),
            pltpu.VMEM_SHARED((VOCAB,), jnp.float32),
            pltpu.SemaphoreType.DMA,
            pltpu.SemaphoreType.DMA,
            pltpu.SemaphoreType.DMA,
            pltpu.SemaphoreType.DMA,
        ],
        compiler_params=pltpu.CompilerParams(use_tc_tiling_on_sc=False),
    )
    def kern(xr_hbm, p_hbm, b_hbm, out_hbm,
             idx_v0, idx_v1, pg_v0, pg_v1, b_v, out_v, p_sh,
             sem_i0, sem_i1, sem_g0, sem_g1):
        wid = lax.axis_index("s") * NC + lax.axis_index("c")
        sid = lax.axis_index("s")
        gbase = wid * N_CHUNKS
        pltpu.sync_copy(b_hbm, b_v)
        bias = b_v[...]

        STAGE = VOCAB // 8

        @pl.when(sid < 8)
        def _stage():
            pltpu.sync_copy(p_hbm.at[pl.ds(sid * STAGE, STAGE)],
                            p_sh.at[pl.ds(sid * STAGE, STAGE)])

        plsc.subcore_barrier()

        def start_idx(g, ibuf, sem):
            pltpu.async_copy(
                xr_hbm.at[pl.ds(g * IDX_PER_CHUNK, IDX_PER_CHUNK)], ibuf, sem)

        def wait_idx(ibuf, sem):
            pltpu.make_async_copy(
                xr_hbm.at[pl.ds(0, IDX_PER_CHUNK)], ibuf, sem).wait()

        def fire_gathers(ibuf, gbuf, sem):
            pltpu.async_copy(p_sh.at[ibuf], gbuf, sem)

        def wait_gathers(gbuf, sem):
            pltpu.make_async_copy(
                p_hbm.at[pl.ds(0, IDX_PER_CHUNK)], gbuf, sem).wait()

        def reduce_store(gbuf, c):
            def red_body(jj, accs):
                a0, a1, a2, a3 = accs
                r0 = jj * 128
                a0 = a0 + gbuf[pl.ds(r0, 16)] + gbuf[pl.ds(r0 + 64, 16)]
                a1 = a1 + gbuf[pl.ds(r0 + 16, 16)] + gbuf[pl.ds(r0 + 80, 16)]
                a2 = a2 + gbuf[pl.ds(r0 + 32, 16)] + gbuf[pl.ds(r0 + 96, 16)]
                a3 = a3 + gbuf[pl.ds(r0 + 48, 16)] + gbuf[pl.ds(r0 + 112, 16)]
                return (a0, a1, a2, a3)
            z = jnp.zeros((16,), jnp.float32)
            a0, a1, a2, a3 = lax.fori_loop(0, HIST // 8, red_body, (z, z, z, z))
            acc = (a0 + a1) + (a2 + a3)
            y = acc / jnp.float32(HIST) + bias
            s = 1.0 / (1.0 + jnp.exp(-y))
            t = s * 10000.0
            r = (t + _ROUND_MAGIC) - _ROUND_MAGIC
            out_v[pl.ds(c * CHUNK, CHUNK)] = r / 10000.0

        start_idx(gbase, idx_v0, sem_i0)
        wait_idx(idx_v0, sem_i0)
        fire_gathers(idx_v0, pg_v0, sem_g0)
        start_idx(gbase + 1, idx_v1, sem_i1)

        def pair_body(t, carry):
            a = 2 * t
            b = 2 * t + 1
            wait_idx(idx_v1, sem_i1)
            wait_gathers(pg_v0, sem_g0)
            fire_gathers(idx_v1, pg_v1, sem_g1)

            @pl.when(t < N_PAIRS - 1)
            def _():
                start_idx(gbase + a + 2, idx_v0, sem_i0)

            reduce_store(pg_v0, a)
            wait_gathers(pg_v1, sem_g1)

            @pl.when(t < N_PAIRS - 1)
            def _():
                wait_idx(idx_v0, sem_i0)
                fire_gathers(idx_v0, pg_v0, sem_g0)
                start_idx(gbase + b + 2, idx_v1, sem_i1)

            reduce_store(pg_v1, b)
            return carry

        lax.fori_loop(0, N_PAIRS, pair_body, 0)
        pltpu.sync_copy(out_v, out_hbm.at[pl.ds(wid * ROWS_PER_W, ROWS_PER_W)])

    return kern


_sc_kern = _make_sc_kernel()


def kernel(x, emb_table, W, b):
    p = _tc_p(emb_table.T, W.reshape(EMBED_DIM, 1))
    x_r = (x.reshape(BATCH // CHUNK, CHUNK, HIST)
             .transpose(0, 2, 1)
             .reshape(BATCH * HIST)
             .astype(jnp.int32))
    b_vec = jnp.broadcast_to(b.astype(jnp.float32), (16,))
    out = _sc_kern(x_r, p, b_vec)
    return out.reshape(BATCH, 1)


# speedup vs baseline: 34.0806x; 1.0237x over previous
import functools

import jax
import jax.numpy as jnp
from jax import lax
from jax.experimental import pallas as pl
from jax.experimental.pallas import tpu as pltpu
from jax.experimental.pallas import tpu_sc as plsc

VOCAB = 1000000
EMBED_DIM = 16
BATCH = 16384
HIST = 200

NC = 2
NS = 16
NW = NC * NS
ROWS_PER_W = BATCH // NW
CHUNK = 16
N_CHUNKS = ROWS_PER_W // CHUNK
N_PAIRS = N_CHUNKS // 2
IDX_PER_CHUNK = CHUNK * HIST
GATHER_BLK = 128
N_GATHERS = IDX_PER_CHUNK // GATHER_BLK

_ROUND_MAGIC = 12582912.0

TC_BLK = 131072
TC_GRID = (VOCAB + TC_BLK - 1) // TC_BLK


def _tc_p_body(t_ref, w_ref, p_ref):
    p_ref[...] = jnp.sum(t_ref[...] * w_ref[...], axis=0)


_tc_p = pl.pallas_call(
    _tc_p_body,
    grid=(TC_GRID,),
    in_specs=[
        pl.BlockSpec((EMBED_DIM, TC_BLK), lambda i: (0, i)),
        pl.BlockSpec((EMBED_DIM, 1), lambda i: (0, 0)),
    ],
    out_specs=pl.BlockSpec((TC_BLK,), lambda i: (i,)),
    out_shape=jax.ShapeDtypeStruct((VOCAB,), jnp.float32),
)


def _make_sc_kernel():
    mesh = plsc.VectorSubcoreMesh(
        core_axis_name="c", subcore_axis_name="s", num_cores=NC, num_subcores=NS
    )

    @functools.partial(
        pl.kernel,
        out_type=jax.ShapeDtypeStruct((BATCH,), jnp.float32),
        mesh=mesh,
        scratch_types=[
            pltpu.VMEM((IDX_PER_CHUNK,), jnp.int32),
            pltpu.VMEM((IDX_PER_CHUNK,), jnp.int32),
            pltpu.VMEM((IDX_PER_CHUNK,), jnp.float32),
            pltpu.VMEM((IDX_PER_CHUNK,), jnp.float32),
            pltpu.VMEM((16,), jnp.float32),
            pltpu.VMEM((ROWS_PER_W,), jnp.float32),
            pltpu.VMEM_SHARED((VOCAB,), jnp.float32),
            pltpu.SemaphoreType.DMA,
            pltpu.SemaphoreType.DMA,
            pltpu.SemaphoreType.DMA,
            pltpu.SemaphoreType.DMA,
        ],
        compiler_params=pltpu.CompilerParams(use_tc_tiling_on_sc=False),
    )
    def kern(xr_hbm, p_hbm, b_hbm, out_hbm,
             idx_v0, idx_v1, pg_v0, pg_v1, b_v, out_v, p_sh,
             sem_i0, sem_i1, sem_g0, sem_g1):
        wid = lax.axis_index("s") * NC + lax.axis_index("c")
        sid = lax.axis_index("s")
        gbase = wid * N_CHUNKS
        pltpu.sync_copy(b_hbm, b_v)
        bias = b_v[...]

        def start_idx(g, ibuf, sem):
            pltpu.async_copy(
                xr_hbm.at[pl.ds(g * IDX_PER_CHUNK, IDX_PER_CHUNK)], ibuf, sem)

        def wait_idx(ibuf, sem):
            pltpu.make_async_copy(
                xr_hbm.at[pl.ds(0, IDX_PER_CHUNK)], ibuf, sem).wait()

        def fire_gathers(ibuf, gbuf, sem):
            pltpu.async_copy(p_sh.at[ibuf], gbuf, sem)

        def wait_gathers(gbuf, sem):
            pltpu.make_async_copy(
                p_hbm.at[pl.ds(0, IDX_PER_CHUNK)], gbuf, sem).wait()

        def reduce_store(gbuf, c):
            def red_body(jj, accs):
                a0, a1, a2, a3 = accs
                r0 = jj * 128
                a0 = a0 + gbuf[pl.ds(r0, 16)] + gbuf[pl.ds(r0 + 64, 16)]
                a1 = a1 + gbuf[pl.ds(r0 + 16, 16)] + gbuf[pl.ds(r0 + 80, 16)]
                a2 = a2 + gbuf[pl.ds(r0 + 32, 16)] + gbuf[pl.ds(r0 + 96, 16)]
                a3 = a3 + gbuf[pl.ds(r0 + 48, 16)] + gbuf[pl.ds(r0 + 112, 16)]
                return (a0, a1, a2, a3)
            z = jnp.zeros((16,), jnp.float32)
            a0, a1, a2, a3 = lax.fori_loop(0, HIST // 8, red_body, (z, z, z, z))
            acc = (a0 + a1) + (a2 + a3)
            y = acc / jnp.float32(HIST) + bias
            s = 1.0 / (1.0 + jnp.exp(-y))
            t = s * 10000.0
            r = (t + _ROUND_MAGIC) - _ROUND_MAGIC
            out_v[pl.ds(c * CHUNK, CHUNK)] = r / 10000.0

        start_idx(gbase, idx_v0, sem_i0)
        start_idx(gbase + 1, idx_v1, sem_i1)

        STAGE = 62496

        @pl.when(sid < 15)
        def _stage():
            pltpu.sync_copy(p_hbm.at[pl.ds(sid * STAGE, STAGE)],
                            p_sh.at[pl.ds(sid * STAGE, STAGE)])

        @pl.when(sid == 15)
        def _stage_tail():
            pltpu.sync_copy(p_hbm.at[pl.ds(15 * STAGE, VOCAB - 15 * STAGE)],
                            p_sh.at[pl.ds(15 * STAGE, VOCAB - 15 * STAGE)])

        plsc.subcore_barrier()

        wait_idx(idx_v0, sem_i0)
        fire_gathers(idx_v0, pg_v0, sem_g0)

        def pair_body(t, carry):
            a = 2 * t
            b = 2 * t + 1
            wait_idx(idx_v1, sem_i1)
            wait_gathers(pg_v0, sem_g0)
            fire_gathers(idx_v1, pg_v1, sem_g1)

            @pl.when(t < N_PAIRS - 1)
            def _():
                start_idx(gbase + a + 2, idx_v0, sem_i0)

            reduce_store(pg_v0, a)
            wait_gathers(pg_v1, sem_g1)

            @pl.when(t < N_PAIRS - 1)
            def _():
                wait_idx(idx_v0, sem_i0)
                fire_gathers(idx_v0, pg_v0, sem_g0)
                start_idx(gbase + b + 2, idx_v1, sem_i1)

            reduce_store(pg_v1, b)
            return carry

        lax.fori_loop(0, N_PAIRS, pair_body, 0)
        pltpu.sync_copy(out_v, out_hbm.at[pl.ds(wid * ROWS_PER_W, ROWS_PER_W)])

    return kern


_sc_kern = _make_sc_kernel()


def kernel(x, emb_table, W, b):
    p = _tc_p(emb_table.T, W.reshape(EMBED_DIM, 1))
    x_r = (x.reshape(BATCH // CHUNK, CHUNK, HIST)
             .transpose(0, 2, 1)
             .reshape(BATCH * HIST)
             .astype(jnp.int32))
    b_vec = jnp.broadcast_to(b.astype(jnp.float32), (16,))
    out = _sc_kern(x_r, p, b_vec)
    return out.reshape(BATCH, 1)


# speedup vs baseline: 61.8022x; 1.8134x over previous
import functools

import jax
import jax.numpy as jnp
from jax import lax
from jax.experimental import pallas as pl
from jax.experimental.pallas import tpu as pltpu
from jax.experimental.pallas import tpu_sc as plsc

VOCAB = 1000000
EMBED_DIM = 16
BATCH = 16384
HIST = 200

NC = 2
NS = 16
NW = NC * NS
ROWS_PER_W = BATCH // NW
CHUNK = 64
N_CHUNKS = ROWS_PER_W // CHUNK
N_PAIRS = N_CHUNKS // 2
IDX_PER_CHUNK = CHUNK * HIST
NGRP = CHUNK // 16

_ROUND_MAGIC = 12582912.0

TC_BLK = 131072
TC_GRID = (VOCAB + TC_BLK - 1) // TC_BLK


def _tc_p_body(t_ref, w_ref, p_ref):
    p_ref[...] = jnp.sum(t_ref[...] * w_ref[...], axis=0)


_tc_p = pl.pallas_call(
    _tc_p_body,
    grid=(TC_GRID,),
    in_specs=[
        pl.BlockSpec((EMBED_DIM, TC_BLK), lambda i: (0, i)),
        pl.BlockSpec((EMBED_DIM, 1), lambda i: (0, 0)),
    ],
    out_specs=pl.BlockSpec((TC_BLK,), lambda i: (i,)),
    out_shape=jax.ShapeDtypeStruct((VOCAB,), jnp.float32),
)


def _make_sc_kernel():
    mesh = plsc.VectorSubcoreMesh(
        core_axis_name="c", subcore_axis_name="s", num_cores=NC, num_subcores=NS
    )

    @functools.partial(
        pl.kernel,
        out_type=jax.ShapeDtypeStruct((BATCH,), jnp.float32),
        mesh=mesh,
        scratch_types=[
            pltpu.VMEM((HIST, CHUNK), jnp.int32),
            pltpu.VMEM((HIST, CHUNK), jnp.int32),
            pltpu.VMEM((IDX_PER_CHUNK,), jnp.float32),
            pltpu.VMEM((IDX_PER_CHUNK,), jnp.float32),
            pltpu.VMEM((16,), jnp.float32),
            pltpu.VMEM((ROWS_PER_W,), jnp.float32),
            pltpu.VMEM_SHARED((VOCAB,), jnp.float32),
            pltpu.SemaphoreType.DMA,
            pltpu.SemaphoreType.DMA,
            pltpu.SemaphoreType.DMA,
            pltpu.SemaphoreType.DMA,
        ],
        compiler_params=pltpu.CompilerParams(use_tc_tiling_on_sc=False),
    )
    def kern(xt_hbm, p_hbm, b_hbm, out_hbm,
             idx_v0, idx_v1, pg_v0, pg_v1, b_v, out_v, p_sh,
             sem_i0, sem_i1, sem_g0, sem_g1):
        wid = lax.axis_index("s") * NC + lax.axis_index("c")
        sid = lax.axis_index("s")
        pltpu.sync_copy(b_hbm, b_v)
        bias = b_v[...]

        def start_idx(c, ibuf, sem):
            b0 = (wid * N_CHUNKS + c) * CHUNK
            pltpu.async_copy(xt_hbm.at[:, pl.ds(b0, CHUNK)], ibuf, sem)

        def wait_idx(ibuf, sem):
            pltpu.make_async_copy(
                xt_hbm.at[:, pl.ds(0, CHUNK)], ibuf, sem).wait()

        def fire_gathers(ibuf, gbuf, sem):
            for j in range(HIST):
                pltpu.async_copy(
                    p_sh.at[ibuf.at[j]],
                    gbuf.at[pl.ds(j * CHUNK, CHUNK)],
                    sem)

        def wait_gathers(gbuf, sem):
            pltpu.make_async_copy(
                p_hbm.at[pl.ds(0, IDX_PER_CHUNK)], gbuf, sem).wait()

        def reduce_store(gbuf, c):
            def red_body(jj, accs):
                accs = list(accs)
                for t in range(4):
                    r0 = (jj * 4 + t) * CHUNK
                    for g in range(NGRP):
                        accs[g] = accs[g] + gbuf[pl.ds(r0 + g * 16, 16)]
                return tuple(accs)
            z = jnp.zeros((16,), jnp.float32)
            accs = lax.fori_loop(0, HIST // 4, red_body, (z,) * NGRP)
            for g in range(NGRP):
                y = accs[g] / jnp.float32(HIST) + bias
                s = 1.0 / (1.0 + jnp.exp(-y))
                t = s * 10000.0
                r = (t + _ROUND_MAGIC) - _ROUND_MAGIC
                out_v[pl.ds(c * CHUNK + g * 16, 16)] = r / 10000.0

        start_idx(0, idx_v0, sem_i0)
        start_idx(1, idx_v1, sem_i1)

        STAGE = 62496

        @pl.when(sid < 15)
        def _stage():
            pltpu.sync_copy(p_hbm.at[pl.ds(sid * STAGE, STAGE)],
                            p_sh.at[pl.ds(sid * STAGE, STAGE)])

        @pl.when(sid == 15)
        def _stage_tail():
            pltpu.sync_copy(p_hbm.at[pl.ds(15 * STAGE, VOCAB - 15 * STAGE)],
                            p_sh.at[pl.ds(15 * STAGE, VOCAB - 15 * STAGE)])

        plsc.subcore_barrier()

        wait_idx(idx_v0, sem_i0)
        fire_gathers(idx_v0, pg_v0, sem_g0)

        def pair_body(t, carry):
            a = 2 * t
            b = 2 * t + 1
            wait_idx(idx_v1, sem_i1)
            wait_gathers(pg_v0, sem_g0)
            fire_gathers(idx_v1, pg_v1, sem_g1)

            @pl.when(t < N_PAIRS - 1)
            def _():
                start_idx(a + 2, idx_v0, sem_i0)

            reduce_store(pg_v0, a)
            wait_gathers(pg_v1, sem_g1)

            @pl.when(t < N_PAIRS - 1)
            def _():
                wait_idx(idx_v0, sem_i0)
                fire_gathers(idx_v0, pg_v0, sem_g0)
                start_idx(b + 2, idx_v1, sem_i1)

            reduce_store(pg_v1, b)
            return carry

        lax.fori_loop(0, N_PAIRS, pair_body, 0)
        pltpu.sync_copy(out_v, out_hbm.at[pl.ds(wid * ROWS_PER_W, ROWS_PER_W)])

    return kern


_sc_kern = _make_sc_kernel()


def kernel(x, emb_table, W, b):
    p = _tc_p(emb_table.T, W.reshape(EMBED_DIM, 1))
    x_t = x.T.astype(jnp.int32)
    b_vec = jnp.broadcast_to(b.astype(jnp.float32), (16,))
    out = _sc_kern(x_t, p, b_vec)
    return out.reshape(BATCH, 1)
